# Initial kernel scaffold; baseline (speedup 1.0000x reference)
import jax
import jax.numpy as jnp
from jax.experimental import pallas as pl


def kernel(features, adj_edge_index, adj_edge_weight, weight):
    raise NotImplementedError("write your pallas kernel here")


# speedup vs baseline: 3.9400x; 3.9400x over previous
import functools

import jax
import jax.numpy as jnp
from jax import lax
from jax.experimental import pallas as pl
from jax.experimental.pallas import tpu as pltpu
from jax.experimental.pallas import tpu_sc as plsc

N_NODES = 10000
N_EDGES = 160000
D_HALF = 128
NC = 2
NS = 16
L = 16

E_PER_TILE = N_EDGES // NS
CHUNK = 80
N_CHUNKS = E_PER_TILE // CHUNK
STAGE = 25
N_STAGE = N_CHUNKS // STAGE
RBLK = 80


def _mm_body(x_ref, w_ref, o_ref):
    o_ref[...] = jnp.dot(x_ref[...], w_ref[...],
                         preferred_element_type=jnp.float32)


def _support_split(features, weight):
    grid = (2, 10)
    return pl.pallas_call(
        _mm_body,
        grid=grid,
        in_specs=[
            pl.BlockSpec((1000, 256), lambda j, i: (i, 0)),
            pl.BlockSpec((256, 128), lambda j, i: (0, j)),
        ],
        out_specs=pl.BlockSpec((1000, 128), lambda j, i: (j * 10 + i, 0)),
        out_shape=jax.ShapeDtypeStruct((2 * N_NODES, D_HALF), jnp.float32),
    )(features, weight)


@functools.partial(
    pl.kernel,
    mesh=plsc.VectorSubcoreMesh(core_axis_name="c", subcore_axis_name="s"),
    out_type=jax.ShapeDtypeStruct((NC, N_NODES, D_HALF), jnp.float32),
    scratch_types=[
        pltpu.VMEM((STAGE, CHUNK), jnp.int32),
        pltpu.VMEM((STAGE, CHUNK), jnp.int32),
        pltpu.VMEM((STAGE, CHUNK), jnp.float32),
        pltpu.VMEM((CHUNK, D_HALF), jnp.float32),
        pltpu.VMEM_SHARED((N_NODES, D_HALF), jnp.float32),
        pltpu.SemaphoreType.DMA,
    ],
)
def _sc_aggregate(support_hbm, src_hbm, dst_hbm, w_hbm, out_hbm,
                  src_v, dst_v, w_v, gbuf, acc, sem):
    c = lax.axis_index("c")
    s = lax.axis_index("s")

    zeros16 = jnp.zeros((L,), jnp.float32)

    def _zero_body(k, _):
        r = k // (D_HALF // L)
        v = k % (D_HALF // L)
        gbuf[r, pl.ds(v * L, L)] = zeros16
        return 0

    lax.fori_loop(0, RBLK * (D_HALF // L), _zero_body, 0)
    base = s * (8 * RBLK)
    nblk = jnp.where(s == NS - 1, 5, 8)

    def _zero_copy(i, _):
        st = pl.multiple_of(base + i * RBLK, 8)
        pltpu.sync_copy(gbuf, acc.at[pl.ds(st, RBLK)])
        return 0

    lax.fori_loop(0, nblk, _zero_copy, 0)
    plsc.subcore_barrier()

    def _stage_body(b, _):
        pltpu.sync_copy(src_hbm.at[c, s, b], src_v)
        pltpu.sync_copy(dst_hbm.at[s, b], dst_v)
        pltpu.sync_copy(w_hbm.at[s, b], w_v)

        def _chunk_body(j, _):
            pltpu.async_copy(support_hbm.at[src_v.at[j]], gbuf, sem).wait()

            def _scale_group(g, _):
                wvec = w_v[j, pl.ds(g * L, L)]
                for e2 in range(L):
                    e = g * L + e2
                    wv = jnp.full((L,), wvec[e2], jnp.float32)
                    for v in range(D_HALF // L):
                        sl = pl.ds(v * L, L)
                        gbuf[e, sl] = gbuf[e, sl] * wv
                return 0

            lax.fori_loop(0, CHUNK // L, _scale_group, 0)
            pltpu.sync_copy(gbuf, acc.at[dst_v.at[j]], add=True)
            return 0

        lax.fori_loop(0, STAGE, _chunk_body, 0)
        return 0

    lax.fori_loop(0, N_STAGE, _stage_body, 0)
    plsc.subcore_barrier()

    def _out_copy(i, _):
        st = pl.multiple_of(base + i * RBLK, 8)
        pltpu.sync_copy(acc.at[pl.ds(st, RBLK)],
                        out_hbm.at[c, pl.ds(st, RBLK)])
        return 0

    lax.fori_loop(0, nblk, _out_copy, 0)


def kernel(features, adj_edge_index, adj_edge_weight, weight):
    dst = adj_edge_index[0].astype(jnp.int32)
    src = adj_edge_index[1].astype(jnp.int32)
    support = _support_split(features, weight)
    src2 = jnp.stack([src, src + N_NODES]).reshape(
        NC, NS, N_STAGE, STAGE, CHUNK)
    dst3 = dst.reshape(NS, N_STAGE, STAGE, CHUNK)
    w3 = adj_edge_weight.reshape(NS, N_STAGE, STAGE, CHUNK)
    out2 = _sc_aggregate(support, src2, dst3, w3)
    return out2.transpose(1, 0, 2).reshape(N_NODES, NC * D_HALF)
